# Initial kernel scaffold; baseline (speedup 1.0000x reference)
import jax
import jax.numpy as jnp
from jax.experimental import pallas as pl


def kernel(x, router_logits, gating_einsum, linear, per_expert_scale, router_scale):
    raise NotImplementedError("write your pallas kernel here")


# speedup vs baseline: 8.4946x; 8.4946x over previous
import jax
import jax.numpy as jnp
from jax.experimental import pallas as pl
from jax.experimental.pallas import tpu as pltpu

_EMBED = 1024
_HIDDEN = 512
_E = 64
_K = 8
_BM = 256
_ROWS = 2048 * _K
_MAX_BLOCKS = _ROWS // _BM + _E


def _ffn_block_kernel(be_ref, xm_ref, vd_ref, x_ref, ge_ref, lin_ref, out_ref):
    i = pl.program_id(0)

    @pl.when(vd_ref[i] == 1)
    def _():
        xb = x_ref[...].astype(jnp.bfloat16)
        g0 = ge_ref[0, 0].astype(jnp.bfloat16)
        g1 = ge_ref[0, 1].astype(jnp.bfloat16)
        x1 = jax.lax.dot_general(xb, g0, (((1,), (1,)), ((), ())),
                                 preferred_element_type=jnp.float32)
        x2 = jax.lax.dot_general(xb, g1, (((1,), (1,)), ((), ())),
                                 preferred_element_type=jnp.float32)
        act = (jax.nn.gelu(x1) * x2).astype(jnp.bfloat16)
        lin = lin_ref[0].astype(jnp.bfloat16)
        out_ref[...] = jax.lax.dot_general(
            act, lin, (((1,), (0,)), ((), ())),
            preferred_element_type=jnp.float32)


def _grouped_ffn(sorted_x, gating_einsum, linear, block_expert, block_xmap, block_valid):
    grid_spec = pltpu.PrefetchScalarGridSpec(
        num_scalar_prefetch=3,
        grid=(_MAX_BLOCKS,),
        in_specs=[
            pl.BlockSpec((_BM, _EMBED), lambda i, be, xm, vd: (xm[i], 0)),
            pl.BlockSpec((1, 2, _HIDDEN, _EMBED), lambda i, be, xm, vd: (be[i], 0, 0, 0)),
            pl.BlockSpec((1, _HIDDEN, _EMBED), lambda i, be, xm, vd: (be[i], 0, 0)),
        ],
        out_specs=pl.BlockSpec((_BM, _EMBED), lambda i, be, xm, vd: (xm[i], 0)),
    )
    return pl.pallas_call(
        _ffn_block_kernel,
        grid_spec=grid_spec,
        out_shape=jax.ShapeDtypeStruct((_MAX_BLOCKS * _BM, _EMBED), jnp.float32),
        compiler_params=pltpu.CompilerParams(
            dimension_semantics=("arbitrary",)),
    )(block_expert, block_xmap, block_valid, sorted_x, gating_einsum, linear)


def kernel(x, router_logits, gating_einsum, linear, per_expert_scale, router_scale):
    g, s, d = x.shape
    t = g * s
    xf = x.reshape(t, d)

    var = jnp.mean(jnp.square(xf), axis=-1, keepdims=True)
    ri = xf * jax.lax.rsqrt(var + 1e-06)
    ri = ri * jax.lax.rsqrt(jnp.float32(d)) * router_scale
    logits = ri @ router_logits
    probs = jax.nn.softmax(logits, axis=-1)
    _, choices = jax.lax.approx_max_k(logits, k=_K)
    indicator = jax.nn.one_hot(choices, _E, dtype=probs.dtype).sum(axis=-2)
    renorm = jnp.sum(indicator * probs, axis=-1, keepdims=True)
    renorm = jnp.where(renorm > 0.0, renorm, 1.0)
    cw = jnp.take_along_axis(probs / renorm, choices, axis=-1)
    cw = cw * per_expert_scale[choices]

    cf = choices.reshape(-1)
    ohi = jax.nn.one_hot(cf, _E, dtype=jnp.int32)
    counts = ohi.sum(axis=0)
    ranks = jnp.take_along_axis(jnp.cumsum(ohi, axis=0), cf[:, None], axis=1)[:, 0] - 1
    blocks = (counts + _BM - 1) // _BM
    cumblocks = jnp.cumsum(blocks)
    used = cumblocks[-1]
    padoff = (jnp.concatenate([jnp.zeros((1,), cumblocks.dtype), cumblocks[:-1]]) * _BM)
    pos = padoff[cf] + ranks
    tok = jnp.arange(_ROWS, dtype=jnp.int32) // _K
    rowtok = jnp.zeros((_MAX_BLOCKS * _BM,), jnp.int32).at[pos].set(tok)

    bidx = jnp.arange(_MAX_BLOCKS, dtype=jnp.int32)
    be = jnp.minimum(jnp.searchsorted(cumblocks, bidx, side="right"), _E - 1).astype(jnp.int32)
    valid = (bidx < used)
    last = (used - 1).astype(jnp.int32)
    be = jnp.where(valid, be, be[last])
    xm = jnp.where(valid, bidx, last)
    vd = valid.astype(jnp.int32)

    sorted_x = xf[rowtok]
    eo = _grouped_ffn(sorted_x, gating_einsum, linear, be, xm, vd)
    coll = eo[pos.reshape(t, _K)]
    out = jnp.einsum('tkd,tk->td', coll, cw)
    return out.reshape(g, s, d)
